# baseline (speedup 1.0000x reference)
import functools

import jax
import jax.numpy as jnp
from jax import lax
from jax.experimental import pallas as pl
from jax.experimental.pallas import tpu as pltpu
from jax.experimental.pallas import tpu_sc as plsc

TARGET = 98304
NROWS = 65536
NUP = TARGET - NROWS
D = 128
L = 16

NC, NS = 2, 16
NW = NC * NS
UP_PER_W = NUP // NW
CP_PER_W = NROWS // NW
GB = 128
NB = UP_PER_W // GB
CC = 64
NCH = CP_PER_W // CC
CPG = NCH // NB

_MESH = plsc.VectorSubcoreMesh(
    core_axis_name="c", subcore_axis_name="s", num_cores=NC, num_subcores=NS
)


@functools.partial(
    pl.kernel,
    out_type=jax.ShapeDtypeStruct((TARGET, D), jnp.float32),
    mesh=_MESH,
    scratch_types=[
        pltpu.VMEM((UP_PER_W,), jnp.int32),
        pltpu.VMEM((UP_PER_W,), jnp.int32),
        [pltpu.VMEM((GB, D), jnp.float32) for _ in range(2)],
        [pltpu.VMEM((GB, D), jnp.float32) for _ in range(2)],
        [pltpu.VMEM((GB, D), jnp.float32) for _ in range(2)],
        [pltpu.VMEM((CC, D), jnp.float32) for _ in range(3)],
        [pltpu.SemaphoreType.DMA for _ in range(2)],
        [pltpu.SemaphoreType.DMA for _ in range(2)],
        [pltpu.SemaphoreType.DMA for _ in range(3)],
        pltpu.SemaphoreType.DMA,
    ],
)
def _hex_unpool(
    x_hbm, idx0_hbm, idx1_hbm, out_hbm, i0v, i1v, r0, r1, ob, cb, sg, so, cs, si
):
    wid = lax.axis_index("s") * NC + lax.axis_index("c")
    ubase = wid * UP_PER_W
    cbase = wid * CP_PER_W

    di0 = pltpu.async_copy(idx0_hbm.at[pl.ds(ubase, UP_PER_W)], i0v, si)
    di1 = pltpu.async_copy(idx1_hbm.at[pl.ds(ubase, UP_PER_W)], i1v, si)

    cl = [None] * 3
    cstores = [None] * 3
    for b in range(3):
        cl[b] = pltpu.async_copy(x_hbm.at[pl.ds(cbase + b * CC, CC)], cb[b], cs[b])

    def copy_step(c):
        bb = c % 3
        cl[bb].wait()
        cstores[bb] = pltpu.async_copy(
            cb[bb], out_hbm.at[pl.ds(cbase + c * CC, CC)], cs[bb]
        )
        f = c + 2
        if 3 <= f < NCH:
            bf = f % 3
            if cstores[bf] is not None:
                cstores[bf].wait()
            cl[bf] = pltpu.async_copy(
                x_hbm.at[pl.ds(cbase + f * CC, CC)], cb[bf], cs[bf]
            )
            cstores[bf] = None

    def start_gathers(j):
        b = j % 2
        isl = pl.ds(j * GB, GB)
        d0 = pltpu.async_copy(x_hbm.at[i0v.at[isl]], r0[b], sg[b])
        d1 = pltpu.async_copy(x_hbm.at[i1v.at[isl]], r1[b], sg[b])
        return d0, d1

    di0.wait()
    di1.wait()
    pend = [start_gathers(0), start_gathers(1)]
    outst = [None, None]

    for j in range(NB):
        b = j % 2
        pend[b][0].wait()
        pend[b][1].wait()

        def avg_body(r, carry, _b=b):
            for c in range(D // L):
                a = r0[_b][r, pl.ds(c * L, L)]
                bb = r1[_b][r, pl.ds(c * L, L)]
                ob[_b][r, pl.ds(c * L, L)] = (a + bb) * 0.5
            return carry

        lax.fori_loop(0, GB, avg_body, 0)

        if j + 2 < NB:
            pend[b] = start_gathers(j + 2)
        if outst[b] is not None:
            outst[b].wait()
        outst[b] = pltpu.async_copy(
            ob[b], out_hbm.at[pl.ds(NROWS + ubase + j * GB, GB)], so[b]
        )

        for c in range(j * CPG, (j + 1) * CPG):
            copy_step(c)

    for st in outst:
        if st is not None:
            st.wait()
    for st in cstores:
        if st is not None:
            st.wait()


def kernel(x, upsample_indices):
    idx0 = upsample_indices[:, 0]
    idx1 = upsample_indices[:, 1]
    return _hex_unpool(x, idx0, idx1)
